# Initial kernel scaffold; baseline (speedup 1.0000x reference)
import jax
import jax.numpy as jnp
from jax.experimental import pallas as pl


def kernel(x, weight, bias):
    raise NotImplementedError("write your pallas kernel here")


# speedup vs baseline: 1.5558x; 1.5558x over previous
import jax
import jax.numpy as jnp
from jax.experimental import pallas as pl
from jax.experimental.pallas import tpu as pltpu


def _conv_kernel(e_ref, o_ref, w_ref, b_ref, out_ref):
    _, C, T_out = out_ref.shape

    e = e_ref[0]
    o = o_ref[0]
    e2 = jnp.concatenate(
        [e[:, 1:], jnp.zeros((C, 1), jnp.bfloat16)], axis=1)

    dn = (((0,), (0,)), ((), ()))
    y = jax.lax.dot_general(w_ref[0], e, dn,
                            preferred_element_type=jnp.float32)
    y += jax.lax.dot_general(w_ref[1], o, dn,
                             preferred_element_type=jnp.float32)
    y += jax.lax.dot_general(w_ref[2], e2, dn,
                             preferred_element_type=jnp.float32)
    y += b_ref[...]
    out_ref[0] = y.astype(out_ref.dtype)


def kernel(x, weight, bias):
    B, C, T = x.shape
    T_out = (T - 2) // 2 + 1
    xp = x.reshape(B, C, T // 2, 2)
    e = xp[..., 0].astype(jnp.bfloat16)
    o = xp[..., 1].astype(jnp.bfloat16)
    w = jnp.transpose(weight, (2, 1, 0)).astype(jnp.bfloat16)
    b = bias.reshape(C, 1)

    out = pl.pallas_call(
        _conv_kernel,
        out_shape=jax.ShapeDtypeStruct((B, C, T_out), x.dtype),
        grid=(B,),
        in_specs=[
            pl.BlockSpec((1, C, T_out), lambda i: (i, 0, 0)),
            pl.BlockSpec((1, C, T_out), lambda i: (i, 0, 0)),
            pl.BlockSpec((3, C, C), lambda i: (0, 0, 0)),
            pl.BlockSpec((C, 1), lambda i: (0, 0)),
        ],
        out_specs=pl.BlockSpec((1, C, T_out), lambda i: (i, 0, 0)),
        compiler_params=pltpu.CompilerParams(
            dimension_semantics=("parallel",)),
    )(e, o, w, b)
    return out


# speedup vs baseline: 5.2909x; 3.4007x over previous
import numpy as np
import jax
import jax.numpy as jnp
from jax.experimental import pallas as pl
from jax.experimental.pallas import tpu as pltpu

_CHUNK = 256


def _conv_kernel(x_ref, p_ref, w_ref, b_ref, out_ref):
    _, C, T = x_ref.shape
    T_out = out_ref.shape[2]
    chunk = p_ref.shape[0]
    half = chunk // 2
    p = p_ref[...]

    evens, odds = [], []
    for j in range(T // chunk):
        pc = x_ref[0, :, chunk * j:chunk * (j + 1)].astype(jnp.bfloat16)
        s = jnp.dot(pc, p,
                    preferred_element_type=jnp.float32).astype(jnp.bfloat16)
        evens.append(s[:, :half])
        odds.append(s[:, half:])
    e = jnp.concatenate(evens, axis=1)
    o = jnp.concatenate(odds, axis=1)
    e2 = jnp.concatenate(
        [e[:, 1:], jnp.zeros((C, 1), jnp.bfloat16)], axis=1)

    dn = (((0,), (0,)), ((), ()))
    y = jax.lax.dot_general(w_ref[0], e, dn,
                            preferred_element_type=jnp.float32)
    y += jax.lax.dot_general(w_ref[1], o, dn,
                             preferred_element_type=jnp.float32)
    y += jax.lax.dot_general(w_ref[2], e2, dn,
                             preferred_element_type=jnp.float32)
    y += b_ref[...]
    out_ref[0] = y.astype(out_ref.dtype)


def _selection_matrix(chunk):
    half = chunk // 2
    p = np.zeros((chunk, chunk), np.float32)
    idx = np.arange(half)
    p[2 * idx, idx] = 1.0
    p[2 * idx + 1, half + idx] = 1.0
    return jnp.asarray(p, jnp.bfloat16)


def kernel(x, weight, bias):
    B, C, T = x.shape
    T_out = (T - 2) // 2 + 1
    chunk = min(_CHUNK, T)
    p = _selection_matrix(chunk)
    w = jnp.transpose(weight, (2, 1, 0)).astype(jnp.bfloat16)
    b = bias.reshape(C, 1)

    out = pl.pallas_call(
        _conv_kernel,
        out_shape=jax.ShapeDtypeStruct((B, C, T_out), x.dtype),
        grid=(B,),
        in_specs=[
            pl.BlockSpec((1, C, T), lambda i: (i, 0, 0)),
            pl.BlockSpec((chunk, chunk), lambda i: (0, 0)),
            pl.BlockSpec((3, C, C), lambda i: (0, 0, 0)),
            pl.BlockSpec((C, 1), lambda i: (0, 0)),
        ],
        out_specs=pl.BlockSpec((1, C, T_out), lambda i: (i, 0, 0)),
        compiler_params=pltpu.CompilerParams(
            dimension_semantics=("parallel",)),
    )(x, p, w, b)
    return out
